# baseline (speedup 1.0000x reference)
import functools

import jax
import jax.numpy as jnp
from jax import lax
from jax.experimental import pallas as pl
from jax.experimental.pallas import tpu as pltpu
from jax.experimental.pallas import tpu_sc as plsc

B, L, DIM = 4, 2048, 4096
NUM_LAYERS, TOP = 24, 5
CK = 256
SPLIT = 2048
LEFT_STEPS = SPLIT // CK
RIGHT_STEPS = (DIM - SPLIT) // CK

NW = 32
CW = 128
NSTRIPES = SPLIT // CW
RB = 512
UNROLL = 8


def _scpool_body(x_hbm, out0_hbm, out1_hbm, buf, accv):
    nc = 2
    wid = lax.axis_index("s") * nc + lax.axis_index("c")
    stripe = wid // 2
    half = wid % 2
    col0 = stripe * CW
    ngroups = CW // 16
    for bi in range(2):
        accs = tuple(jnp.zeros((16,), jnp.float32) for _ in range(ngroups))
        for blk in range(L // RB):
            row0 = (half * 2 + bi) * L + blk * RB
            pltpu.sync_copy(
                x_hbm.at[pl.ds(row0, RB), pl.ds(col0, CW)], buf)

            def body(i, carry):
                a = list(carry)
                base = i * UNROLL
                for r in range(UNROLL):
                    for g in range(ngroups):
                        a[g] = a[g] + buf[base + r, pl.ds(g * 16, 16)]
                return tuple(a)

            accs = lax.fori_loop(0, RB // UNROLL, body, accs)
        for g in range(ngroups):
            accv[bi, pl.ds(g * 16, 16)] = accs[g] * (1.0 / L)

    @pl.when(half == 0)
    def _():
        pltpu.sync_copy(accv, out0_hbm.at[pl.ds(0, 2), pl.ds(col0, CW)])

    @pl.when(half == 1)
    def _():
        pltpu.sync_copy(accv, out1_hbm.at[pl.ds(0, 2), pl.ds(col0, CW)])


def _sc_pool(x2d):
    mesh = plsc.VectorSubcoreMesh(core_axis_name="c", subcore_axis_name="s")
    fn = functools.partial(
        pl.kernel,
        mesh=mesh,
        out_type=[
            jax.ShapeDtypeStruct((2, SPLIT), jnp.float32),
            jax.ShapeDtypeStruct((2, SPLIT), jnp.float32),
        ],
        scratch_types=[
            pltpu.VMEM((RB, CW), jnp.float32),
            pltpu.VMEM((2, CW), jnp.float32),
        ],
    )(_scpool_body)
    return fn(x2d)


def _tc_right_body(x_ref, w1_ref, w2_ref, acc1_ref, acc2_ref):
    i = pl.program_id(0)
    pooled_c = jnp.sum(x_ref[...], axis=1) * (1.0 / L)
    p1 = lax.dot_general(pooled_c, w1_ref[...], (((1,), (1,)), ((), ())),
                         preferred_element_type=jnp.float32)
    p2 = lax.dot_general(pooled_c, w2_ref[...], (((1,), (1,)), ((), ())),
                         preferred_element_type=jnp.float32)

    @pl.when(i == 0)
    def _():
        acc1_ref[...] = p1
        acc2_ref[...] = p2

    @pl.when(i > 0)
    def _():
        acc1_ref[...] += p1
        acc2_ref[...] += p2


def _tc_final_body(p0_ref, p1_ref, w1_ref, w2_ref, accr1_ref, accr2_ref,
                   w3_ref, b1_ref, b2_ref, b3_ref,
                   idx_ref, wts_ref, probs_ref, acc1_ref, acc2_ref):
    i = pl.program_id(0)
    pooled_c = jnp.concatenate([p0_ref[...], p1_ref[...]], axis=0)
    p1 = lax.dot_general(pooled_c, w1_ref[...], (((1,), (1,)), ((), ())),
                         preferred_element_type=jnp.float32)
    p2 = lax.dot_general(pooled_c, w2_ref[...], (((1,), (1,)), ((), ())),
                         preferred_element_type=jnp.float32)

    @pl.when(i == 0)
    def _():
        acc1_ref[...] = accr1_ref[...] + p1
        acc2_ref[...] = accr2_ref[...] + p2

    @pl.when(i > 0)
    def _():
        acc1_ref[...] += p1
        acc2_ref[...] += p2

    @pl.when(i == LEFT_STEPS - 1)
    def _epilogue():
        h1 = jax.nn.silu(acc1_ref[...] + b1_ref[...])
        h2 = jax.nn.silu(acc2_ref[...] + b2_ref[...])
        gated = h1 * h2
        logits = lax.dot_general(gated, w3_ref[...], (((1,), (1,)), ((), ())),
                                 preferred_element_type=jnp.float32)
        logits = logits + b3_ref[...]

        m = jnp.max(logits, axis=-1, keepdims=True)
        e = jnp.exp(logits - m)
        probs = e / jnp.sum(e, axis=-1, keepdims=True)
        probs_ref[...] = probs

        iota = lax.broadcasted_iota(jnp.int32, (B, NUM_LAYERS), 1)
        work = probs
        idx_cols = []
        wt_cols = []
        for _ in range(TOP):
            mv = jnp.max(work, axis=-1, keepdims=True)
            sel = jnp.min(jnp.where(work == mv, iota, NUM_LAYERS * 2),
                          axis=-1, keepdims=True)
            idx_cols.append(sel)
            wt_cols.append(mv)
            work = jnp.where(iota == sel, -jnp.inf, work)
        top_w = jnp.concatenate(wt_cols, axis=1)
        idx_ref[...] = jnp.concatenate(idx_cols, axis=1)
        wts_ref[...] = top_w / jnp.sum(top_w, axis=-1, keepdims=True)


@jax.jit
def _router(text_features, W1, W2, W3, b1, b2, b3):
    x2d = text_features.reshape(B * L, DIM)
    pooled_l0, pooled_l1 = _sc_pool(x2d)

    acc1r, acc2r = pl.pallas_call(
        _tc_right_body,
        grid=(RIGHT_STEPS,),
        in_specs=[
            pl.BlockSpec((B, L, CK), lambda i: (0, 0, i + LEFT_STEPS)),
            pl.BlockSpec((DIM, CK), lambda i: (0, i + LEFT_STEPS)),
            pl.BlockSpec((DIM, CK), lambda i: (0, i + LEFT_STEPS)),
        ],
        out_specs=[
            pl.BlockSpec((B, DIM), lambda i: (0, 0)),
            pl.BlockSpec((B, DIM), lambda i: (0, 0)),
        ],
        out_shape=[
            jax.ShapeDtypeStruct((B, DIM), jnp.float32),
            jax.ShapeDtypeStruct((B, DIM), jnp.float32),
        ],
        compiler_params=pltpu.CompilerParams(
            dimension_semantics=("arbitrary",),
        ),
    )(text_features, W1, W2)

    outs = pl.pallas_call(
        _tc_final_body,
        grid=(LEFT_STEPS,),
        in_specs=[
            pl.BlockSpec((2, CK), lambda i: (0, i)),
            pl.BlockSpec((2, CK), lambda i: (0, i)),
            pl.BlockSpec((DIM, CK), lambda i: (0, i)),
            pl.BlockSpec((DIM, CK), lambda i: (0, i)),
            pl.BlockSpec((B, DIM), lambda i: (0, 0)),
            pl.BlockSpec((B, DIM), lambda i: (0, 0)),
            pl.BlockSpec((NUM_LAYERS, DIM), lambda i: (0, 0)),
            pl.BlockSpec((1, DIM), lambda i: (0, 0)),
            pl.BlockSpec((1, DIM), lambda i: (0, 0)),
            pl.BlockSpec((1, NUM_LAYERS), lambda i: (0, 0)),
        ],
        out_specs=[
            pl.BlockSpec((B, TOP), lambda i: (0, 0)),
            pl.BlockSpec((B, TOP), lambda i: (0, 0)),
            pl.BlockSpec((B, NUM_LAYERS), lambda i: (0, 0)),
        ],
        out_shape=[
            jax.ShapeDtypeStruct((B, TOP), jnp.int32),
            jax.ShapeDtypeStruct((B, TOP), jnp.float32),
            jax.ShapeDtypeStruct((B, NUM_LAYERS), jnp.float32),
        ],
        scratch_shapes=[
            pltpu.VMEM((B, DIM), jnp.float32),
            pltpu.VMEM((B, DIM), jnp.float32),
        ],
        compiler_params=pltpu.CompilerParams(
            dimension_semantics=("arbitrary",),
        ),
    )(pooled_l0, pooled_l1, W1, W2, acc1r, acc2r, W3,
      b1[None, :], b2[None, :], b3[None, :])
    return outs


def kernel(text_features, W1, b1, W2, b2, W3, b3):
    top_i, top_w, probs = _router(text_features, W1, W2, W3, b1, b2, b3)
    return (top_i, top_w, probs)


# speedup vs baseline: 1.2489x; 1.2489x over previous
import jax
import jax.numpy as jnp
from jax import lax
from jax.experimental import pallas as pl
from jax.experimental.pallas import tpu as pltpu

B, L, DIM = 4, 2048, 4096
NUM_LAYERS, TOP = 24, 5
LC = 256
LSTEPS = L // LC
CK = 256
KSTEPS = DIM // CK


def _router_body(x_ref, w1_ref, w2_ref, w3_ref, b1_ref, b2_ref, b3_ref,
                 idx_ref, wts_ref, probs_ref, accp_ref, acc1_ref, acc2_ref):
    i = pl.program_id(0)

    @pl.when(i < LSTEPS)
    def _pool():
        part = jnp.sum(x_ref[...], axis=1)
        for k in range(KSTEPS):
            chunk = part[:, k * CK:(k + 1) * CK]

            @pl.when(i == 0)
            def _():
                accp_ref[k] = chunk

            @pl.when(i > 0)
            def _():
                accp_ref[k] += chunk

    @pl.when(i >= LSTEPS)
    def _matmul():
        k = i - LSTEPS
        pooled_c = accp_ref[k] * (1.0 / L)
        p1 = lax.dot_general(pooled_c, w1_ref[...], (((1,), (1,)), ((), ())),
                             preferred_element_type=jnp.float32)
        p2 = lax.dot_general(pooled_c, w2_ref[...], (((1,), (1,)), ((), ())),
                             preferred_element_type=jnp.float32)

        @pl.when(k == 0)
        def _():
            acc1_ref[...] = p1
            acc2_ref[...] = p2

        @pl.when(k > 0)
        def _():
            acc1_ref[...] += p1
            acc2_ref[...] += p2

    @pl.when(i == LSTEPS + KSTEPS - 1)
    def _epilogue():
        h1 = jax.nn.silu(acc1_ref[...] + b1_ref[...])
        h2 = jax.nn.silu(acc2_ref[...] + b2_ref[...])
        gated = h1 * h2
        logits = lax.dot_general(gated, w3_ref[...], (((1,), (1,)), ((), ())),
                                 preferred_element_type=jnp.float32)
        logits = logits + b3_ref[...]

        m = jnp.max(logits, axis=-1, keepdims=True)
        e = jnp.exp(logits - m)
        probs = e / jnp.sum(e, axis=-1, keepdims=True)
        probs_ref[...] = probs

        iota = lax.broadcasted_iota(jnp.int32, (B, NUM_LAYERS), 1)
        work = probs
        idx_cols = []
        wt_cols = []
        for _ in range(TOP):
            mv = jnp.max(work, axis=-1, keepdims=True)
            sel = jnp.min(jnp.where(work == mv, iota, NUM_LAYERS * 2),
                          axis=-1, keepdims=True)
            idx_cols.append(sel)
            wt_cols.append(mv)
            work = jnp.where(iota == sel, -jnp.inf, work)
        top_w = jnp.concatenate(wt_cols, axis=1)
        idx_ref[...] = jnp.concatenate(idx_cols, axis=1)
        wts_ref[...] = top_w / jnp.sum(top_w, axis=-1, keepdims=True)


@jax.jit
def _router(text_features, W1, W2, W3, b1, b2, b3):
    grid = (LSTEPS + KSTEPS,)
    kernel_fn = pl.pallas_call(
        _router_body,
        grid=grid,
        in_specs=[
            pl.BlockSpec((B, LC, DIM),
                         lambda i: (0, jnp.minimum(i, LSTEPS - 1), 0)),
            pl.BlockSpec((DIM, CK), lambda i: (0, jnp.maximum(i - LSTEPS, 0))),
            pl.BlockSpec((DIM, CK), lambda i: (0, jnp.maximum(i - LSTEPS, 0))),
            pl.BlockSpec((NUM_LAYERS, DIM), lambda i: (0, 0)),
            pl.BlockSpec((1, DIM), lambda i: (0, 0)),
            pl.BlockSpec((1, DIM), lambda i: (0, 0)),
            pl.BlockSpec((1, NUM_LAYERS), lambda i: (0, 0)),
        ],
        out_specs=[
            pl.BlockSpec((B, TOP), lambda i: (0, 0)),
            pl.BlockSpec((B, TOP), lambda i: (0, 0)),
            pl.BlockSpec((B, NUM_LAYERS), lambda i: (0, 0)),
        ],
        out_shape=[
            jax.ShapeDtypeStruct((B, TOP), jnp.int32),
            jax.ShapeDtypeStruct((B, TOP), jnp.float32),
            jax.ShapeDtypeStruct((B, NUM_LAYERS), jnp.float32),
        ],
        scratch_shapes=[
            pltpu.VMEM((KSTEPS, B, CK), jnp.float32),
            pltpu.VMEM((B, DIM), jnp.float32),
            pltpu.VMEM((B, DIM), jnp.float32),
        ],
        compiler_params=pltpu.CompilerParams(
            dimension_semantics=("arbitrary",),
        ),
    )
    return kernel_fn(text_features, W1, W2, W3,
                     b1[None, :], b2[None, :], b3[None, :])


def kernel(text_features, W1, b1, W2, b2, W3, b3):
    top_i, top_w, probs = _router(text_features, W1, W2, W3, b1, b2, b3)
    return (top_i, top_w, probs)


# speedup vs baseline: 1.2813x; 1.0259x over previous
import functools

import jax
import jax.numpy as jnp
from jax.experimental import pallas as pl
from jax.experimental.pallas import tpu as pltpu

B, L, DIM = 4, 2048, 4096
NUM_LAYERS, TOP = 24, 5
CK = 256
KSTEPS = DIM // CK


def _router_body(x_ref, w1_ref, w2_ref, w3_ref, b1_ref, b2_ref, b3_ref,
                 idx_ref, wts_ref, probs_ref, acc1_ref, acc2_ref):
    i = pl.program_id(0)

    pooled_c = jnp.sum(x_ref[...], axis=1) * (1.0 / L)

    p1 = jax.lax.dot_general(pooled_c, w1_ref[...],
                             (((1,), (1,)), ((), ())),
                             preferred_element_type=jnp.float32)
    p2 = jax.lax.dot_general(pooled_c, w2_ref[...],
                             (((1,), (1,)), ((), ())),
                             preferred_element_type=jnp.float32)

    @pl.when(i == 0)
    def _init():
        acc1_ref[...] = p1
        acc2_ref[...] = p2

    @pl.when(i > 0)
    def _acc():
        acc1_ref[...] += p1
        acc2_ref[...] += p2

    @pl.when(i == KSTEPS - 1)
    def _epilogue():
        h1 = jax.nn.silu(acc1_ref[...] + b1_ref[...])
        h2 = jax.nn.silu(acc2_ref[...] + b2_ref[...])
        gated = h1 * h2
        logits = jax.lax.dot_general(gated, w3_ref[...],
                                     (((1,), (1,)), ((), ())),
                                     preferred_element_type=jnp.float32)
        logits = logits + b3_ref[...]

        m = jnp.max(logits, axis=-1, keepdims=True)
        e = jnp.exp(logits - m)
        probs = e / jnp.sum(e, axis=-1, keepdims=True)
        probs_ref[...] = probs

        iota = jax.lax.broadcasted_iota(jnp.int32, (B, NUM_LAYERS), 1)
        work = probs
        idx_cols = []
        wt_cols = []
        for _ in range(TOP):
            mv = jnp.max(work, axis=-1, keepdims=True)
            is_max = work == mv
            sel = jnp.min(jnp.where(is_max, iota, NUM_LAYERS * 2),
                          axis=-1, keepdims=True)
            idx_cols.append(sel)
            wt_cols.append(mv)
            work = jnp.where(iota == sel, -jnp.inf, work)
        top_w = jnp.concatenate(wt_cols, axis=1)
        top_i = jnp.concatenate(idx_cols, axis=1)
        idx_ref[...] = top_i
        wts_ref[...] = top_w / jnp.sum(top_w, axis=-1, keepdims=True)


@jax.jit
def _router(text_features, W1, W2, W3, b1, b2, b3):
    grid = (KSTEPS,)
    kernel_fn = pl.pallas_call(
        _router_body,
        grid=grid,
        in_specs=[
            pl.BlockSpec((B, L, CK), lambda i: (0, 0, i)),
            pl.BlockSpec((DIM, CK), lambda i: (0, i)),
            pl.BlockSpec((DIM, CK), lambda i: (0, i)),
            pl.BlockSpec((NUM_LAYERS, DIM), lambda i: (0, 0)),
            pl.BlockSpec((1, DIM), lambda i: (0, 0)),
            pl.BlockSpec((1, DIM), lambda i: (0, 0)),
            pl.BlockSpec((1, NUM_LAYERS), lambda i: (0, 0)),
        ],
        out_specs=[
            pl.BlockSpec((B, TOP), lambda i: (0, 0)),
            pl.BlockSpec((B, TOP), lambda i: (0, 0)),
            pl.BlockSpec((B, NUM_LAYERS), lambda i: (0, 0)),
        ],
        out_shape=[
            jax.ShapeDtypeStruct((B, TOP), jnp.int32),
            jax.ShapeDtypeStruct((B, TOP), jnp.float32),
            jax.ShapeDtypeStruct((B, NUM_LAYERS), jnp.float32),
        ],
        scratch_shapes=[
            pltpu.VMEM((B, DIM), jnp.float32),
            pltpu.VMEM((B, DIM), jnp.float32),
        ],
        compiler_params=pltpu.CompilerParams(
            dimension_semantics=("arbitrary",),
        ),
    )
    return kernel_fn(text_features, W1, W2, W3,
                     b1[None, :], b2[None, :], b3[None, :])


def kernel(text_features, W1, b1, W2, b2, W3, b3):
    top_i, top_w, probs = _router(text_features, W1, W2, W3, b1, b2, b3)
    return (top_i, top_w, probs)
